# baseline (speedup 1.0000x reference)
import functools

import jax
import jax.numpy as jnp
from jax import lax
from jax.experimental import pallas as pl
from jax.experimental.pallas import tpu as pltpu
from jax.experimental.pallas import tpu_sc as plsc

_N = 7
_M = _N * (_N - 1) // 2
_LANES = 16
_THRESHOLDS = tuple((i * (i - 1)) // 2 for i in range(2, _N))


def _body(x_hbm, vec_hbm, out_hbm, x_v, gat_v, sem):
    only_tile0 = (lax.axis_index("c") == 0) & (lax.axis_index("s") == 0)

    @pl.when(only_tile0)
    def _():
        pltpu.sync_copy(x_hbm, x_v)
        xv = x_v[...]

        idx = jnp.int32(0)
        for chunk in range(2):
            k = jnp.arange(_LANES, dtype=jnp.int32) + jnp.int32(chunk * _LANES)
            row = jnp.full((_LANES,), 1, jnp.int32)
            for t in _THRESHOLDS:
                row = row + jnp.where(k >= t, jnp.int32(1), jnp.int32(0))
            tri = lax.shift_right_logical(row * (row - 1), jnp.int32(1))
            col = k - tri
            valid = k < _M
            row = jnp.where(valid, row, jnp.int32(0))
            col = jnp.where(valid, col, jnp.int32(0))
            a = plsc.load_gather(x_v, [row])
            b = plsc.load_gather(x_v, [col])
            w = jnp.where(valid, lax.shift_left(jnp.int32(1), k), jnp.int32(0))
            idx = idx + jnp.sum(jnp.where(a == b, w, jnp.int32(0)))
        del xv

        idx_vec = jnp.broadcast_to(idx, (_LANES,))
        pltpu.async_copy(vec_hbm.at[idx_vec], gat_v, sem).wait()
        pltpu.sync_copy(gat_v, out_hbm)


@jax.jit
def kernel(x, vec):
    mesh = plsc.VectorSubcoreMesh(
        core_axis_name="c", subcore_axis_name="s", num_cores=2, num_subcores=16
    )
    run = functools.partial(
        pl.kernel,
        mesh=mesh,
        out_type=jax.ShapeDtypeStruct((_LANES,), jnp.float32),
        scratch_types=[
            pltpu.VMEM((_LANES,), jnp.int32),
            pltpu.VMEM((_LANES,), jnp.float32),
            pltpu.SemaphoreType.DMA,
        ],
        compiler_params=pltpu.CompilerParams(needs_layout_passes=False),
    )(_body)
    xp = jnp.zeros((_LANES,), jnp.int32).at[:_N].set(x.astype(jnp.int32))
    out = run(xp, vec)
    return out[0]


# speedup vs baseline: 1.0069x; 1.0069x over previous
import functools

import jax
import jax.numpy as jnp
from jax import lax
from jax.experimental import pallas as pl
from jax.experimental.pallas import tpu as pltpu
from jax.experimental.pallas import tpu_sc as plsc

_N = 7
_M = _N * (_N - 1) // 2
_LANES = 16
_THRESHOLDS = tuple((i * (i - 1)) // 2 for i in range(2, _N))


def _body(x_hbm, vec_hbm, out_hbm, x_v, gat_v, sem):
    only_tile0 = (lax.axis_index("c") == 0) & (lax.axis_index("s") == 0)

    @pl.when(only_tile0)
    def _():
        pltpu.sync_copy(x_hbm, x_v.at[pl.ds(0, _N)])

        idx = jnp.int32(0)
        for chunk in range(2):
            k = jnp.arange(_LANES, dtype=jnp.int32) + jnp.int32(chunk * _LANES)
            row = jnp.full((_LANES,), 1, jnp.int32)
            for t in _THRESHOLDS:
                row = row + jnp.where(k >= t, jnp.int32(1), jnp.int32(0))
            tri = lax.shift_right_logical(row * (row - 1), jnp.int32(1))
            col = k - tri
            valid = k < _M
            row = jnp.where(valid, row, jnp.int32(0))
            col = jnp.where(valid, col, jnp.int32(0))
            a = plsc.load_gather(x_v, [row])
            b = plsc.load_gather(x_v, [col])
            w = jnp.where(valid, lax.shift_left(jnp.int32(1), k), jnp.int32(0))
            idx = idx + jnp.sum(jnp.where(a == b, w, jnp.int32(0)))

        idx_vec = jnp.broadcast_to(idx, (_LANES,))
        pltpu.async_copy(vec_hbm.at[idx_vec], gat_v, sem).wait()
        pltpu.sync_copy(gat_v.at[pl.ds(0, 1)], out_hbm)


@jax.jit
def kernel(x, vec):
    mesh = plsc.VectorSubcoreMesh(
        core_axis_name="c", subcore_axis_name="s", num_cores=2, num_subcores=16
    )
    run = functools.partial(
        pl.kernel,
        mesh=mesh,
        out_type=jax.ShapeDtypeStruct((1,), jnp.float32),
        scratch_types=[
            pltpu.VMEM((_LANES,), jnp.int32),
            pltpu.VMEM((_LANES,), jnp.float32),
            pltpu.SemaphoreType.DMA,
        ],
        compiler_params=pltpu.CompilerParams(
            needs_layout_passes=False,
            skip_device_barrier=True,
            disable_bounds_checks=True,
            disable_semaphore_checks=True,
        ),
    )(_body)
    out = run(x.astype(jnp.int32), vec)
    return out.reshape(())


# speedup vs baseline: 1.0764x; 1.0690x over previous
import functools

import jax
import jax.numpy as jnp
from jax import lax
from jax.experimental import pallas as pl
from jax.experimental.pallas import tpu as pltpu
from jax.experimental.pallas import tpu_sc as plsc

_N = 7
_M = _N * (_N - 1) // 2
_LANES = 16
_THRESHOLDS = tuple((i * (i - 1)) // 2 for i in range(2, _N))


def _body(x_hbm, vec_hbm, out_hbm, x_v, gat_v, sem):
    only_tile0 = (lax.axis_index("c") == 0) & (lax.axis_index("s") == 0)

    @pl.when(only_tile0)
    def _():
        pltpu.sync_copy(x_hbm, x_v.at[pl.ds(0, _N)])

        idx = jnp.int32(0)
        for chunk in range(2):
            k = jnp.arange(_LANES, dtype=jnp.int32) + jnp.int32(chunk * _LANES)
            row = jnp.full((_LANES,), 1, jnp.int32)
            for t in _THRESHOLDS:
                row = row + jnp.where(k >= t, jnp.int32(1), jnp.int32(0))
            tri = lax.shift_right_logical(row * (row - 1), jnp.int32(1))
            col = k - tri
            valid = k < _M
            row = jnp.where(valid, row, jnp.int32(0))
            col = jnp.where(valid, col, jnp.int32(0))
            a = plsc.load_gather(x_v, [row])
            b = plsc.load_gather(x_v, [col])
            w = jnp.where(valid, lax.shift_left(jnp.int32(1), k), jnp.int32(0))
            idx = idx + jnp.sum(jnp.where(a == b, w, jnp.int32(0)))

        idx_vec = jnp.broadcast_to(idx, (_LANES,))
        pltpu.async_copy(vec_hbm.at[idx_vec], gat_v, sem).wait()
        pltpu.sync_copy(gat_v.at[pl.ds(0, 1)], out_hbm)


@jax.jit
def kernel(x, vec):
    mesh = plsc.VectorSubcoreMesh(
        core_axis_name="c", subcore_axis_name="s", num_cores=1, num_subcores=16
    )
    run = functools.partial(
        pl.kernel,
        mesh=mesh,
        out_type=jax.ShapeDtypeStruct((1,), jnp.float32),
        scratch_types=[
            pltpu.VMEM((_LANES,), jnp.int32),
            pltpu.VMEM((_LANES,), jnp.float32),
            pltpu.SemaphoreType.DMA,
        ],
        compiler_params=pltpu.CompilerParams(
            needs_layout_passes=False,
            skip_device_barrier=True,
            disable_bounds_checks=True,
            disable_semaphore_checks=True,
        ),
    )(_body)
    out = run(x.astype(jnp.int32), vec)
    return out.reshape(())


# speedup vs baseline: 1.1720x; 1.0888x over previous
import functools

import jax
import jax.numpy as jnp
from jax import lax
from jax.experimental import pallas as pl
from jax.experimental.pallas import tpu as pltpu
from jax.experimental.pallas import tpu_sc as plsc

_N = 7
_M = _N * (_N - 1) // 2


def _body(x_hbm, vec_hbm, out_hbm, x_s, buf_s, res_s):
    pltpu.sync_copy(x_hbm, x_s)
    xs = [x_s[i] for i in range(_N)]
    idx = jnp.int32(0)
    k = 0
    for i in range(1, _N):
        for j in range(i):
            idx = idx + jnp.where(xs[i] == xs[j], jnp.int32(1 << k), jnp.int32(0))
            k += 1
    pltpu.sync_copy(vec_hbm.at[idx >> 7], buf_s)
    res_s[0] = buf_s[idx & jnp.int32(127)]
    pltpu.sync_copy(res_s, out_hbm)


@jax.jit
def kernel(x, vec):
    mesh = plsc.ScalarSubcoreMesh(axis_name="c", num_cores=1)
    run = functools.partial(
        pl.kernel,
        mesh=mesh,
        out_type=jax.ShapeDtypeStruct((8,), jnp.float32),
        scratch_types=[
            pltpu.SMEM((8,), jnp.int32),
            pltpu.SMEM((128,), jnp.float32),
            pltpu.SMEM((8,), jnp.float32),
        ],
        compiler_params=pltpu.CompilerParams(
            needs_layout_passes=False,
            skip_device_barrier=True,
            disable_bounds_checks=True,
            disable_semaphore_checks=True,
        ),
    )(_body)
    xp = jnp.zeros((8,), jnp.int32).at[:_N].set(x.astype(jnp.int32))
    out = run(xp, vec.reshape(2 ** _M // 128, 128))
    return out[0]
